# Initial kernel scaffold; baseline (speedup 1.0000x reference)
import jax
import jax.numpy as jnp
from jax.experimental import pallas as pl


def kernel(input, is_training):
    raise NotImplementedError("write your pallas kernel here")


# speedup vs baseline: 10.8482x; 10.8482x over previous
import functools

import jax
import jax.numpy as jnp
from jax import lax
from jax.experimental import pallas as pl

B, F = 128, 32768
K = F // 2
ROWS_PER_BLK = 16


def _threshold_body(x_ref, t_ref):
    a = jnp.abs(x_ref[...])
    bits = lax.bitcast_convert_type(a, jnp.int32)
    r = jnp.zeros((ROWS_PER_BLK, 1), jnp.int32)
    for i in range(30, -1, -1):
        cand = r | (1 << i)
        cnt = jnp.sum((bits < cand).astype(jnp.int32), axis=1, keepdims=True)
        r = jnp.where(cnt <= K, cand, r)
    t_ref[...] = lax.bitcast_convert_type(r, jnp.float32)


def _thresholds(x):
    return pl.pallas_call(
        _threshold_body,
        grid=(B // ROWS_PER_BLK,),
        in_specs=[pl.BlockSpec((ROWS_PER_BLK, F), lambda i: (i, 0))],
        out_specs=pl.BlockSpec((ROWS_PER_BLK, 1), lambda i: (i, 0)),
        out_shape=jax.ShapeDtypeStruct((B, 1), jnp.float32),
    )(x)


def _eval_body(x_ref, t_ref, o_ref):
    a = jnp.abs(x_ref[...])
    t = t_ref[...]
    m = jnp.where(a > t, a, 0.0)
    o_ref[...] = m.T.reshape(1, 256, B)


def _eval_out(x, t):
    out = pl.pallas_call(
        _eval_body,
        grid=(B,),
        in_specs=[
            pl.BlockSpec((B, 256), lambda b: (0, b)),
            pl.BlockSpec((B, 1), lambda b: (0, 0)),
        ],
        out_specs=pl.BlockSpec((1, 256, B), lambda b: (b, 0, 0)),
        out_shape=jax.ShapeDtypeStruct((B, 256, B), jnp.float32),
    )(x, t)
    return out.reshape(B, F)


def _train_body(x_ref, t_ref, m2_ref, o_ref):
    a = jnp.abs(x_ref[...])
    t = t_ref[...]
    drop = (a <= t) & (m2_ref[...] != 0)
    o_ref[...] = jnp.where(drop, 0.0, a)


def _train_out(x, t):
    u = jax.random.uniform(
        jax.random.fold_in(jax.random.key(0), 1), (F, B), dtype=jnp.float32)
    m2 = (u <= 0.5).T.astype(jnp.float32)
    return pl.pallas_call(
        _train_body,
        grid=(B // ROWS_PER_BLK,),
        in_specs=[
            pl.BlockSpec((ROWS_PER_BLK, F), lambda i: (i, 0)),
            pl.BlockSpec((ROWS_PER_BLK, 1), lambda i: (i, 0)),
            pl.BlockSpec((ROWS_PER_BLK, F), lambda i: (i, 0)),
        ],
        out_specs=pl.BlockSpec((ROWS_PER_BLK, F), lambda i: (i, 0)),
        out_shape=jax.ShapeDtypeStruct((B, F), jnp.float32),
    )(x, t, m2)


def kernel(input, is_training):
    x = input.reshape(B, F)

    def eval_branch(x):
        return _eval_out(x, _thresholds(x))

    def train_branch(x):
        return _train_out(x, _thresholds(x))

    out = lax.cond(jnp.asarray(is_training) == 0, eval_branch, train_branch, x)
    return out.reshape(input.shape)
